# baseline (speedup 1.0000x reference)
import jax
import jax.numpy as jnp
from jax.experimental import pallas as pl
from jax.experimental.pallas import tpu as pltpu
from jax.experimental.pallas import tpu_sc as plsc

_EMB = 128
_LIN = 256
_GW = 256
_BM = 12800
_CHUNK_BLOCKS = (2, 4, 5, 5)


def _matmul_relu_block(x_ref, w_ref, b_ref, o_ref):
    acc = jnp.dot(x_ref[...], w_ref[...], preferred_element_type=jnp.float32)
    o_ref[...] = jnp.maximum(acc + b_ref[...], 0.0)


def _matmul_relu_block_aliased(x_ref, w_ref, b_ref, buf_ref, o_ref):
    del buf_ref
    acc = jnp.dot(x_ref[...], w_ref[...], preferred_element_type=jnp.float32)
    o_ref[...] = jnp.maximum(acc + b_ref[...], 0.0)


def kernel(inputs, table, W1, b1):
    B, L = inputs.shape
    n = B * L
    idx = inputs.T.reshape(1, n).astype(jnp.int32)

    mesh = plsc.VectorSubcoreMesh(core_axis_name="core", subcore_axis_name="subcore")

    def gather_chunk(row_base, rows):
        base = row_base // _GW

        @pl.kernel(out_type=jax.ShapeDtypeStruct((rows, _EMB), table.dtype), mesh=mesh)
        def gather_rows(table_hbm, idx_hbm, out_hbm):
            def body(idx_vmem, out_vmem):
                pltpu.sync_copy(table_hbm.at[idx_vmem.at[0]], out_vmem)

            pltpu.emit_pipeline(
                body,
                grid=(rows // _GW,),
                in_specs=[pl.BlockSpec((1, _GW), index_map=lambda i: (0, base + i))],
                out_specs=[pl.BlockSpec((_GW, _EMB), index_map=lambda i: (i, 0))],
                core_axis_name=("core", "subcore"),
                dimension_semantics=(pltpu.PARALLEL,),
            )(idx_hbm, out_hbm)

        return gather_rows(table, idx)

    embs = []
    row_base = 0
    for nblocks in _CHUNK_BLOCKS:
        embs.append(gather_chunk(row_base, nblocks * _BM))
        row_base += nblocks * _BM

    w_spec = pl.BlockSpec((_EMB, _LIN), lambda i: (0, 0))
    b_spec = pl.BlockSpec((1, _LIN), lambda i: (0, 0))
    b2d = b1.reshape(1, _LIN)
    buf = None
    block_base = 0
    for nblocks, emb in zip(_CHUNK_BLOCKS, embs):
        base = block_base
        block_base += nblocks
        blocks_per_chunk = nblocks
        out_spec = pl.BlockSpec((_BM, _LIN), lambda i, base=base: (base + i, 0))
        if buf is None:
            buf = pl.pallas_call(
                _matmul_relu_block,
                grid=(blocks_per_chunk,),
                in_specs=[pl.BlockSpec((_BM, _EMB), lambda i: (i, 0)), w_spec, b_spec],
                out_specs=out_spec,
                out_shape=jax.ShapeDtypeStruct((n, _LIN), jnp.float32),
            )(emb, W1, b2d)
        else:
            buf = pl.pallas_call(
                _matmul_relu_block_aliased,
                grid=(blocks_per_chunk,),
                in_specs=[
                    pl.BlockSpec((_BM, _EMB), lambda i: (i, 0)),
                    w_spec,
                    b_spec,
                    pl.BlockSpec(memory_space=pltpu.MemorySpace.HBM),
                ],
                out_specs=out_spec,
                out_shape=jax.ShapeDtypeStruct((n, _LIN), jnp.float32),
                input_output_aliases={3: 0},
            )(emb, W1, b2d, buf)

    return buf.reshape(L, B, _LIN).transpose(1, 0, 2)


# speedup vs baseline: 1.0174x; 1.0174x over previous
import jax
import jax.numpy as jnp
from jax.experimental import pallas as pl
from jax.experimental.pallas import tpu as pltpu
from jax.experimental.pallas import tpu_sc as plsc

_EMB = 128
_LIN = 256
_GW = 256
_BM = 12800
_CHUNK_BLOCKS = (4, 4, 4, 4)


def _matmul_relu_block(x_ref, w_ref, b_ref, o_ref):
    acc = jnp.dot(x_ref[...], w_ref[...], preferred_element_type=jnp.float32)
    o_ref[...] = jnp.maximum(acc + b_ref[...], 0.0)


def _matmul_relu_block_aliased(x_ref, w_ref, b_ref, buf_ref, o_ref):
    del buf_ref
    acc = jnp.dot(x_ref[...], w_ref[...], preferred_element_type=jnp.float32)
    o_ref[...] = jnp.maximum(acc + b_ref[...], 0.0)


def kernel(inputs, table, W1, b1):
    B, L = inputs.shape
    n = B * L
    idx = inputs.T.reshape(1, n).astype(jnp.int32)

    mesh = plsc.VectorSubcoreMesh(core_axis_name="core", subcore_axis_name="subcore")

    def gather_chunk(row_base, rows):
        base = row_base // _GW

        @pl.kernel(out_type=jax.ShapeDtypeStruct((rows, _EMB), table.dtype), mesh=mesh)
        def gather_rows(table_hbm, idx_hbm, out_hbm):
            def body(idx_vmem, out_vmem):
                pltpu.sync_copy(table_hbm.at[idx_vmem.at[0]], out_vmem)

            pltpu.emit_pipeline(
                body,
                grid=(rows // _GW,),
                in_specs=[pl.BlockSpec((1, _GW), index_map=lambda i: (0, base + i))],
                out_specs=[pl.BlockSpec((_GW, _EMB), index_map=lambda i: (i, 0))],
                core_axis_name=("core", "subcore"),
                dimension_semantics=(pltpu.PARALLEL,),
            )(idx_hbm, out_hbm)

        return gather_rows(table, idx)

    embs = []
    row_base = 0
    for nblocks in _CHUNK_BLOCKS:
        embs.append(gather_chunk(row_base, nblocks * _BM))
        row_base += nblocks * _BM

    w_spec = pl.BlockSpec((_EMB, _LIN), lambda i: (0, 0))
    b_spec = pl.BlockSpec((1, _LIN), lambda i: (0, 0))
    b2d = b1.reshape(1, _LIN)
    buf = None
    block_base = 0
    for nblocks, emb in zip(_CHUNK_BLOCKS, embs):
        base = block_base
        block_base += nblocks
        blocks_per_chunk = nblocks
        out_spec = pl.BlockSpec((_BM, _LIN), lambda i, base=base: (base + i, 0))
        if buf is None:
            buf = pl.pallas_call(
                _matmul_relu_block,
                grid=(blocks_per_chunk,),
                in_specs=[pl.BlockSpec((_BM, _EMB), lambda i: (i, 0)), w_spec, b_spec],
                out_specs=out_spec,
                out_shape=jax.ShapeDtypeStruct((n, _LIN), jnp.float32),
            )(emb, W1, b2d)
        else:
            buf = pl.pallas_call(
                _matmul_relu_block_aliased,
                grid=(blocks_per_chunk,),
                in_specs=[
                    pl.BlockSpec((_BM, _EMB), lambda i: (i, 0)),
                    w_spec,
                    b_spec,
                    pl.BlockSpec(memory_space=pltpu.MemorySpace.HBM),
                ],
                out_specs=out_spec,
                out_shape=jax.ShapeDtypeStruct((n, _LIN), jnp.float32),
                input_output_aliases={3: 0},
            )(emb, W1, b2d, buf)

    return buf.reshape(L, B, _LIN).transpose(1, 0, 2)
